# initial kernel scaffold (unmeasured)
import jax
import jax.numpy as jnp
from jax import lax
from jax.experimental import pallas as pl
from jax.experimental.pallas import tpu as pltpu


def kernel(
    x,
):
    def body(*refs):
        pass

    out_shape = jax.ShapeDtypeStruct(..., jnp.float32)
    return pl.pallas_call(body, out_shape=out_shape)(...)


# baseline (device time: 224608 ns/iter reference)
import jax
import jax.numpy as jnp
from jax import lax
from jax.experimental import pallas as pl
from jax.experimental.pallas import tpu as pltpu


def kernel(x):
    m, n = x.shape

    def body(x_ref, out_ref, send_sem, recv_sem):
        my_x = lax.axis_index("x")
        my_y = lax.axis_index("y")
        my_z = lax.axis_index("z")
        peer = (my_x, 1 - my_y, my_z)

        barrier_sem = pltpu.get_barrier_semaphore()
        pl.semaphore_signal(
            barrier_sem, inc=1,
            device_id=peer, device_id_type=pl.DeviceIdType.MESH,
        )
        pl.semaphore_wait(barrier_sem, 1)

        out_ref[pl.ds(my_y * m, m), :] = x_ref[...]

        rdma = pltpu.make_async_remote_copy(
            src_ref=x_ref,
            dst_ref=out_ref.at[pl.ds(my_y * m, m), :],
            send_sem=send_sem,
            recv_sem=recv_sem,
            device_id=peer,
            device_id_type=pl.DeviceIdType.MESH,
        )
        rdma.start()
        rdma.wait()

    return pl.pallas_call(
        body,
        out_shape=jax.ShapeDtypeStruct((2 * m, n), x.dtype),
        in_specs=[pl.BlockSpec(memory_space=pltpu.VMEM)],
        out_specs=pl.BlockSpec(memory_space=pltpu.VMEM),
        scratch_shapes=[
            pltpu.SemaphoreType.DMA,
            pltpu.SemaphoreType.DMA,
        ],
        compiler_params=pltpu.CompilerParams(collective_id=0),
    )(x)


# device time: 142628 ns/iter; 1.5748x vs baseline; 1.5748x over previous
import jax
import jax.numpy as jnp
from jax import lax
from jax.experimental import pallas as pl
from jax.experimental.pallas import tpu as pltpu

C = 16


def kernel(x):
    m, n = x.shape
    half = m // 2
    rows = half // C

    def body(x_ref, out_ref, ysend, yrecv, zsend, zrecv):
        my_x = lax.axis_index("x")
        my_y = lax.axis_index("y")
        my_z = lax.axis_index("z")
        peer_y = (my_x, 1 - my_y, my_z)
        peer_z = (my_x, my_y, 1 - my_z)

        barrier_sem = pltpu.get_barrier_semaphore()
        for nbr in (peer_y, peer_z):
            pl.semaphore_signal(
                barrier_sem, inc=1,
                device_id=nbr, device_id_type=pl.DeviceIdType.MESH,
            )
        pl.semaphore_wait(barrier_sem, 2)

        send_base = my_y * m + my_z * half
        recv_base = (1 - my_y) * m + my_z * half

        def y_rdma(k):
            return pltpu.make_async_remote_copy(
                src_ref=x_ref.at[pl.ds(my_z * half + k * rows, rows), :],
                dst_ref=out_ref.at[pl.ds(send_base + k * rows, rows), :],
                send_sem=ysend.at[k],
                recv_sem=yrecv.at[k],
                device_id=peer_y,
                device_id_type=pl.DeviceIdType.MESH,
            )

        def z_rdma(k):
            return pltpu.make_async_remote_copy(
                src_ref=out_ref.at[pl.ds(recv_base + k * rows, rows), :],
                dst_ref=out_ref.at[pl.ds(recv_base + k * rows, rows), :],
                send_sem=zsend.at[k],
                recv_sem=zrecv.at[k],
                device_id=peer_z,
                device_id_type=pl.DeviceIdType.MESH,
            )

        for k in range(C):
            y_rdma(k).start()

        out_ref[pl.ds(my_y * m, m), :] = x_ref[...]

        for k in range(C):
            y_rdma(k).wait_recv()
            z_rdma(k).start()

        for k in range(C):
            z_rdma(k).wait_recv()
        for k in range(C):
            y_rdma(k).wait_send()
            z_rdma(k).wait_send()

    return pl.pallas_call(
        body,
        out_shape=jax.ShapeDtypeStruct((2 * m, n), x.dtype),
        in_specs=[pl.BlockSpec(memory_space=pltpu.VMEM)],
        out_specs=pl.BlockSpec(memory_space=pltpu.VMEM),
        scratch_shapes=[
            pltpu.SemaphoreType.DMA((C,)),
            pltpu.SemaphoreType.DMA((C,)),
            pltpu.SemaphoreType.DMA((C,)),
            pltpu.SemaphoreType.DMA((C,)),
        ],
        compiler_params=pltpu.CompilerParams(collective_id=0),
    )(x)


# device time: 131970 ns/iter; 1.7020x vs baseline; 1.0808x over previous
import jax
import jax.numpy as jnp
from jax import lax
from jax.experimental import pallas as pl
from jax.experimental.pallas import tpu as pltpu

C = 16


def kernel(x):
    m, n = x.shape
    half = m // 2
    rows = half // C

    def body(x_ref, out_ref, ysend, yrecv, zsend, zrecv, copy_sem):
        my_x = lax.axis_index("x")
        my_y = lax.axis_index("y")
        my_z = lax.axis_index("z")
        peer_y = (my_x, 1 - my_y, my_z)
        peer_z = (my_x, my_y, 1 - my_z)

        barrier_sem = pltpu.get_barrier_semaphore()
        for nbr in (peer_y, peer_z):
            pl.semaphore_signal(
                barrier_sem, inc=1,
                device_id=nbr, device_id_type=pl.DeviceIdType.MESH,
            )
        pl.semaphore_wait(barrier_sem, 2)

        send_base = my_y * m + my_z * half
        recv_base = (1 - my_y) * m + my_z * half

        def y_rdma(k):
            return pltpu.make_async_remote_copy(
                src_ref=x_ref.at[pl.ds(my_z * half + k * rows, rows), :],
                dst_ref=out_ref.at[pl.ds(send_base + k * rows, rows), :],
                send_sem=ysend.at[k],
                recv_sem=yrecv.at[k],
                device_id=peer_y,
                device_id_type=pl.DeviceIdType.MESH,
            )

        def z_rdma(k):
            return pltpu.make_async_remote_copy(
                src_ref=out_ref.at[pl.ds(recv_base + k * rows, rows), :],
                dst_ref=out_ref.at[pl.ds(recv_base + k * rows, rows), :],
                send_sem=zsend.at[k],
                recv_sem=zrecv.at[k],
                device_id=peer_z,
                device_id_type=pl.DeviceIdType.MESH,
            )

        for k in range(C):
            y_rdma(k).start()

        local = pltpu.make_async_copy(
            x_ref, out_ref.at[pl.ds(my_y * m, m), :], copy_sem
        )
        local.start()

        for k in range(C):
            y_rdma(k).wait_recv()
            z_rdma(k).start()

        for k in range(C):
            z_rdma(k).wait_recv()
        for k in range(C):
            y_rdma(k).wait_send()
            z_rdma(k).wait_send()
        local.wait()

    return pl.pallas_call(
        body,
        out_shape=jax.ShapeDtypeStruct((2 * m, n), x.dtype),
        in_specs=[pl.BlockSpec(memory_space=pl.ANY)],
        out_specs=pl.BlockSpec(memory_space=pl.ANY),
        scratch_shapes=[
            pltpu.SemaphoreType.DMA((C,)),
            pltpu.SemaphoreType.DMA((C,)),
            pltpu.SemaphoreType.DMA((C,)),
            pltpu.SemaphoreType.DMA((C,)),
            pltpu.SemaphoreType.DMA,
        ],
        compiler_params=pltpu.CompilerParams(collective_id=0),
    )(x)


# device time: 111232 ns/iter; 2.0193x vs baseline; 1.1864x over previous
import jax
import jax.numpy as jnp
from jax import lax
from jax.experimental import pallas as pl
from jax.experimental.pallas import tpu as pltpu

CQ = 8


def kernel(x):
    m, n = x.shape
    q = m // 4
    ch = q // CQ
    hf = CQ // 2

    def body(x_ref, out_ref, ysend, yrecv, xsend_d, xrecv_d, zsend_d,
             zrecv_d, xsend_h, xrecv_h, zsend_h, zrecv_h, copy_sem):
        my_x = lax.axis_index("x")
        my_y = lax.axis_index("y")
        my_z = lax.axis_index("z")
        peer_y = (my_x, 1 - my_y, my_z)
        peer_x = (1 - my_x, my_y, my_z)
        peer_z = (my_x, my_y, 1 - my_z)

        o_mine = (2 * my_x + my_z) * q
        o_xn = (2 * (1 - my_x) + my_z) * q
        o_zn = (2 * my_x + (1 - my_z)) * q
        b_send = my_y * m
        b_recv = (1 - my_y) * m

        barrier_sem = pltpu.get_barrier_semaphore()
        for nbr in (peer_y, peer_x, peer_z):
            pl.semaphore_signal(
                barrier_sem, inc=1,
                device_id=nbr, device_id_type=pl.DeviceIdType.MESH,
            )
        pl.semaphore_wait(barrier_sem, 3)

        def rdma(src_row, dst_row, nrows, send, recv, dev):
            return pltpu.make_async_remote_copy(
                src_ref=out_ref.at[pl.ds(src_row, nrows), :],
                dst_ref=out_ref.at[pl.ds(dst_row, nrows), :],
                send_sem=send, recv_sem=recv,
                device_id=dev, device_id_type=pl.DeviceIdType.MESH,
            )

        def y_rdma(k):
            return pltpu.make_async_remote_copy(
                src_ref=x_ref.at[pl.ds(o_mine + k * ch, ch), :],
                dst_ref=out_ref.at[pl.ds(b_send + o_mine + k * ch, ch), :],
                send_sem=ysend.at[k], recv_sem=yrecv.at[k],
                device_id=peer_y, device_id_type=pl.DeviceIdType.MESH,
            )

        def x_dir(k):
            r = b_recv + o_mine + k * ch
            return rdma(r, r, ch, xsend_d.at[k], xrecv_d.at[k], peer_x)

        def z_dir(k):
            r = b_recv + o_mine + k * ch
            return rdma(r, r, ch, zsend_d.at[k], zrecv_d.at[k], peer_z)

        def x_half(j):
            r = b_recv + o_zn + j * ch
            return rdma(r, r, ch, xsend_h.at[j], xrecv_h.at[j], peer_x)

        def z_half(j):
            r = b_recv + o_xn + (hf + j) * ch
            return rdma(r, r, ch, zsend_h.at[j], zrecv_h.at[j], peer_z)

        for k in range(CQ):
            y_rdma(k).start()

        local = pltpu.make_async_copy(
            x_ref, out_ref.at[pl.ds(b_send, m), :], copy_sem
        )
        local.start()

        for k in range(CQ):
            y_rdma(k).wait_recv()
            x_dir(k).start()
            z_dir(k).start()

        for k in range(CQ):
            z_dir(k).wait_recv()
            if k < hf:
                x_half(k).start()
            x_dir(k).wait_recv()
            if k >= hf:
                z_half(k - hf).start()

        for j in range(hf):
            x_half(j).wait_recv()
            z_half(j).wait_recv()

        for k in range(CQ):
            y_rdma(k).wait_send()
            x_dir(k).wait_send()
            z_dir(k).wait_send()
        for j in range(hf):
            x_half(j).wait_send()
            z_half(j).wait_send()
        local.wait()

    return pl.pallas_call(
        body,
        out_shape=jax.ShapeDtypeStruct((2 * m, n), x.dtype),
        in_specs=[pl.BlockSpec(memory_space=pl.ANY)],
        out_specs=pl.BlockSpec(memory_space=pl.ANY),
        scratch_shapes=[
            pltpu.SemaphoreType.DMA((CQ,)),
            pltpu.SemaphoreType.DMA((CQ,)),
            pltpu.SemaphoreType.DMA((CQ,)),
            pltpu.SemaphoreType.DMA((CQ,)),
            pltpu.SemaphoreType.DMA((CQ,)),
            pltpu.SemaphoreType.DMA((CQ,)),
            pltpu.SemaphoreType.DMA((CQ // 2,)),
            pltpu.SemaphoreType.DMA((CQ // 2,)),
            pltpu.SemaphoreType.DMA((CQ // 2,)),
            pltpu.SemaphoreType.DMA((CQ // 2,)),
            pltpu.SemaphoreType.DMA,
        ],
        compiler_params=pltpu.CompilerParams(collective_id=0),
    )(x)


# device time: 109020 ns/iter; 2.0602x vs baseline; 1.0203x over previous
import jax
import jax.numpy as jnp
from jax import lax
from jax.experimental import pallas as pl
from jax.experimental.pallas import tpu as pltpu

CQ = 16


def kernel(x):
    m, n = x.shape
    q = m // 4
    ch = q // CQ
    hf = CQ // 2

    def body(x_ref, out_ref, ysend, yrecv, xsend_d, xrecv_d, zsend_d,
             zrecv_d, xsend_h, xrecv_h, zsend_h, zrecv_h, copy_sem):
        my_x = lax.axis_index("x")
        my_y = lax.axis_index("y")
        my_z = lax.axis_index("z")
        peer_y = (my_x, 1 - my_y, my_z)
        peer_x = (1 - my_x, my_y, my_z)
        peer_z = (my_x, my_y, 1 - my_z)

        o_mine = (2 * my_x + my_z) * q
        o_xn = (2 * (1 - my_x) + my_z) * q
        o_zn = (2 * my_x + (1 - my_z)) * q
        b_send = my_y * m
        b_recv = (1 - my_y) * m

        barrier_sem = pltpu.get_barrier_semaphore()
        for nbr in (peer_y, peer_x, peer_z):
            pl.semaphore_signal(
                barrier_sem, inc=1,
                device_id=nbr, device_id_type=pl.DeviceIdType.MESH,
            )
        pl.semaphore_wait(barrier_sem, 3)

        def rdma(src_row, dst_row, nrows, send, recv, dev):
            return pltpu.make_async_remote_copy(
                src_ref=out_ref.at[pl.ds(src_row, nrows), :],
                dst_ref=out_ref.at[pl.ds(dst_row, nrows), :],
                send_sem=send, recv_sem=recv,
                device_id=dev, device_id_type=pl.DeviceIdType.MESH,
            )

        def y_rdma(k):
            return pltpu.make_async_remote_copy(
                src_ref=x_ref.at[pl.ds(o_mine + k * ch, ch), :],
                dst_ref=out_ref.at[pl.ds(b_send + o_mine + k * ch, ch), :],
                send_sem=ysend.at[k], recv_sem=yrecv.at[k],
                device_id=peer_y, device_id_type=pl.DeviceIdType.MESH,
            )

        def x_dir(k):
            r = b_recv + o_mine + k * ch
            return rdma(r, r, ch, xsend_d.at[k], xrecv_d.at[k], peer_x)

        def z_dir(k):
            r = b_recv + o_mine + k * ch
            return rdma(r, r, ch, zsend_d.at[k], zrecv_d.at[k], peer_z)

        def x_half(j):
            r = b_recv + o_zn + j * ch
            return rdma(r, r, ch, xsend_h.at[j], xrecv_h.at[j], peer_x)

        def z_half(j):
            r = b_recv + o_xn + (hf + j) * ch
            return rdma(r, r, ch, zsend_h.at[j], zrecv_h.at[j], peer_z)

        for k in range(CQ):
            y_rdma(k).start()

        local = pltpu.make_async_copy(
            x_ref, out_ref.at[pl.ds(b_send, m), :], copy_sem
        )
        local.start()

        for k in range(CQ):
            y_rdma(k).wait_recv()
            x_dir(k).start()
            z_dir(k).start()

        for k in range(CQ):
            z_dir(k).wait_recv()
            if k < hf:
                x_half(k).start()
            x_dir(k).wait_recv()
            if k >= hf:
                z_half(k - hf).start()

        for j in range(hf):
            x_half(j).wait_recv()
            z_half(j).wait_recv()

        for k in range(CQ):
            y_rdma(k).wait_send()
            x_dir(k).wait_send()
            z_dir(k).wait_send()
        for j in range(hf):
            x_half(j).wait_send()
            z_half(j).wait_send()
        local.wait()

    return pl.pallas_call(
        body,
        out_shape=jax.ShapeDtypeStruct((2 * m, n), x.dtype),
        in_specs=[pl.BlockSpec(memory_space=pl.ANY)],
        out_specs=pl.BlockSpec(memory_space=pl.ANY),
        scratch_shapes=[
            pltpu.SemaphoreType.DMA((CQ,)),
            pltpu.SemaphoreType.DMA((CQ,)),
            pltpu.SemaphoreType.DMA((CQ,)),
            pltpu.SemaphoreType.DMA((CQ,)),
            pltpu.SemaphoreType.DMA((CQ,)),
            pltpu.SemaphoreType.DMA((CQ,)),
            pltpu.SemaphoreType.DMA((CQ // 2,)),
            pltpu.SemaphoreType.DMA((CQ // 2,)),
            pltpu.SemaphoreType.DMA((CQ // 2,)),
            pltpu.SemaphoreType.DMA((CQ // 2,)),
            pltpu.SemaphoreType.DMA,
        ],
        compiler_params=pltpu.CompilerParams(collective_id=0),
    )(x)


# device time: 100679 ns/iter; 2.2309x vs baseline; 1.0828x over previous
import jax
import jax.numpy as jnp
from jax import lax
from jax.experimental import pallas as pl
from jax.experimental.pallas import tpu as pltpu

CQ = 16
YG = 6
XH_LO, XH_HI = YG, 11
ZH_LO, ZH_HI = 11, CQ


def kernel(x):
    m, n = x.shape
    q = m // 4
    ch = q // CQ
    nxh = XH_HI - XH_LO
    nzh = ZH_HI - ZH_LO

    def body(x_ref, out_ref, ysend_d, yrecv_d, ysend_g, yrecv_g,
             xsend_d, xrecv_d, zsend_d, zrecv_d,
             xsend_h, xrecv_h, zsend_h, zrecv_h, copy_sem):
        my_x = lax.axis_index("x")
        my_y = lax.axis_index("y")
        my_z = lax.axis_index("z")
        peer_y = (my_x, 1 - my_y, my_z)
        peer_x = (1 - my_x, my_y, my_z)
        peer_z = (my_x, my_y, 1 - my_z)

        o_mine = (2 * my_x + my_z) * q
        o_xn = (2 * (1 - my_x) + my_z) * q
        o_zn = (2 * my_x + (1 - my_z)) * q
        o_diag = (2 * (1 - my_x) + (1 - my_z)) * q
        b_send = my_y * m
        b_recv = (1 - my_y) * m

        barrier_sem = pltpu.get_barrier_semaphore()
        for nbr in (peer_y, peer_x, peer_z):
            pl.semaphore_signal(
                barrier_sem, inc=1,
                device_id=nbr, device_id_type=pl.DeviceIdType.MESH,
            )
        pl.semaphore_wait(barrier_sem, 3)

        def fwd(row, send, recv, dev):
            return pltpu.make_async_remote_copy(
                src_ref=out_ref.at[pl.ds(row, ch), :],
                dst_ref=out_ref.at[pl.ds(row, ch), :],
                send_sem=send, recv_sem=recv,
                device_id=dev, device_id_type=pl.DeviceIdType.MESH,
            )

        def y_send(off, k, send, recv):
            return pltpu.make_async_remote_copy(
                src_ref=x_ref.at[pl.ds(off + k * ch, ch), :],
                dst_ref=out_ref.at[pl.ds(b_send + off + k * ch, ch), :],
                send_sem=send, recv_sem=recv,
                device_id=peer_y, device_id_type=pl.DeviceIdType.MESH,
            )

        def y_dir(k):
            return y_send(o_mine, k, ysend_d.at[k], yrecv_d.at[k])

        def y_diag(k):
            return y_send(o_diag, k, ysend_g.at[k], yrecv_g.at[k])

        def x_dir(k):
            return fwd(b_recv + o_mine + k * ch,
                       xsend_d.at[k], xrecv_d.at[k], peer_x)

        def z_dir(k):
            return fwd(b_recv + o_mine + k * ch,
                       zsend_d.at[k], zrecv_d.at[k], peer_z)

        def x_half(j):
            return fwd(b_recv + o_zn + (XH_LO + j) * ch,
                       xsend_h.at[j], xrecv_h.at[j], peer_x)

        def z_half(j):
            return fwd(b_recv + o_xn + (ZH_LO + j) * ch,
                       zsend_h.at[j], zrecv_h.at[j], peer_z)

        for k in range(CQ):
            y_dir(k).start()
        for k in range(YG):
            y_diag(k).start()

        local = pltpu.make_async_copy(
            x_ref, out_ref.at[pl.ds(b_send, m), :], copy_sem
        )
        local.start()

        for k in range(CQ):
            y_dir(k).wait_recv()
            x_dir(k).start()
            z_dir(k).start()

        for k in range(CQ):
            z_dir(k).wait_recv()
            if XH_LO <= k < XH_HI:
                x_half(k - XH_LO).start()
            x_dir(k).wait_recv()
            if ZH_LO <= k < ZH_HI:
                z_half(k - ZH_LO).start()

        for k in range(YG):
            y_diag(k).wait_recv()
        for j in range(nxh):
            x_half(j).wait_recv()
        for j in range(nzh):
            z_half(j).wait_recv()

        for k in range(CQ):
            y_dir(k).wait_send()
            x_dir(k).wait_send()
            z_dir(k).wait_send()
        for k in range(YG):
            y_diag(k).wait_send()
        for j in range(nxh):
            x_half(j).wait_send()
        for j in range(nzh):
            z_half(j).wait_send()
        local.wait()

    return pl.pallas_call(
        body,
        out_shape=jax.ShapeDtypeStruct((2 * m, n), x.dtype),
        in_specs=[pl.BlockSpec(memory_space=pl.ANY)],
        out_specs=pl.BlockSpec(memory_space=pl.ANY),
        scratch_shapes=[
            pltpu.SemaphoreType.DMA((CQ,)),
            pltpu.SemaphoreType.DMA((CQ,)),
            pltpu.SemaphoreType.DMA((YG,)),
            pltpu.SemaphoreType.DMA((YG,)),
            pltpu.SemaphoreType.DMA((CQ,)),
            pltpu.SemaphoreType.DMA((CQ,)),
            pltpu.SemaphoreType.DMA((CQ,)),
            pltpu.SemaphoreType.DMA((CQ,)),
            pltpu.SemaphoreType.DMA((XH_HI - XH_LO,)),
            pltpu.SemaphoreType.DMA((XH_HI - XH_LO,)),
            pltpu.SemaphoreType.DMA((CQ - ZH_LO,)),
            pltpu.SemaphoreType.DMA((CQ - ZH_LO,)),
            pltpu.SemaphoreType.DMA,
        ],
        compiler_params=pltpu.CompilerParams(collective_id=0),
    )(x)
